# baseline (speedup 1.0000x reference)
import functools

import jax
import jax.numpy as jnp
from jax import lax
from jax.experimental import pallas as pl
from jax.experimental.pallas import tpu as pltpu
from jax.experimental.pallas import tpu_sc as plsc

_PRIME = 5099
_E = 8
_TM = 256
_MAX_TILES = 15
_GROWS = 4096
_IB = 896
_NW = 32


def _nt(a, b):
    return lax.dot_general(a.astype(jnp.bfloat16), b.astype(jnp.bfloat16),
                           (((1,), (1,)), ((), ())),
                           preferred_element_type=jnp.float32)


def _pre_body(hid_ref, sh_ref, tmk_ref, tmr_ref, wr_ref, wks_ref, wvs_ref,
              hk_ref, r_ref, s_ref):
    k = pl.program_id(1)
    hid = hid_ref[...]
    delta = sh_ref[...] - hid
    hk = hid + delta * tmk_ref[...]

    @pl.when(k == 0)
    def _():
        hk_ref[...] = hk
        rin = hid + delta * tmr_ref[...]
        r_ref[...] = jax.nn.sigmoid(_nt(rin, wr_ref[...]))

    h = jnp.maximum(_nt(hk, wks_ref[...]), 0.0)
    h = h * h
    contrib = _nt(h, wvs_ref[...])

    @pl.when(k == 0)
    def _():
        s_ref[...] = contrib

    @pl.when(k > 0)
    def _():
        s_ref[...] += contrib


def _grouped_body(te_ref, x_ref, wk_ref, wv_ref, o_ref):
    k = pl.program_id(1)
    h = jnp.maximum(_nt(x_ref[...], wk_ref[0]), 0.0)
    h = h * h
    contrib = _nt(h, wv_ref[0])

    @pl.when(k == 0)
    def _():
        o_ref[...] = contrib

    @pl.when(k > 0)
    def _():
        o_ref[...] += contrib


def _combine_body(r_ref, s_ref, eo_ref, o_ref):
    o_ref[...] = r_ref[...] * (s_ref[...] + eo_ref[...])


def _sc_gather_rows(table, idx, n_rows, d):
    bpw = n_rows // _NW
    mesh = plsc.VectorSubcoreMesh(core_axis_name="c", subcore_axis_name="s")

    @functools.partial(
        pl.kernel, mesh=mesh,
        out_type=jax.ShapeDtypeStruct((n_rows, d), jnp.float32),
        scratch_types=[
            pltpu.VMEM((bpw,), jnp.int32),
            pltpu.VMEM((bpw, d), jnp.float32),
            pltpu.SemaphoreType.DMA,
        ],
    )
    def k(table_hbm, idx_hbm, out_hbm, idx_v, rows_v, sem):
        wid = lax.axis_index("s") * 2 + lax.axis_index("c")
        base = wid * bpw
        pltpu.sync_copy(idx_hbm.at[pl.ds(base, bpw)], idx_v)
        pltpu.async_copy(table_hbm.at[idx_v], rows_v, sem).wait()
        pltpu.sync_copy(rows_v, out_hbm.at[pl.ds(base, bpw)])

    return k(table, idx)


def kernel(hidden, input_ids, time_maa_k, time_maa_r, W_r,
           Wk_shared, Wv_shared, Wk_experts, Wv_experts):
    B, S, H = hidden.shape
    I = Wk_shared.shape[0]
    K = I // _IB
    M = S // _TM
    P = _MAX_TILES

    x = hidden.reshape(S, H)
    shifted = jnp.pad(x, ((1, 0), (0, 0)))[:-1, :]
    tmk = time_maa_k.reshape(1, H)
    tmr = time_maa_r.reshape(1, H)

    ids = input_ids.reshape(-1).astype(jnp.int32)
    e_t = (ids * _PRIME) % _E
    onehot = (e_t[:, None] == jnp.arange(_E, dtype=jnp.int32)[None, :])
    ranks = jnp.cumsum(onehot.astype(jnp.int32), axis=0)
    counts = ranks[-1]
    rank_t = jnp.take_along_axis(ranks, e_t[:, None], axis=1)[:, 0] - 1
    tiles_per_e = (counts + _TM - 1) // _TM
    tile_end = jnp.cumsum(tiles_per_e)
    tile_start = tile_end - tiles_per_e
    dest = tile_start[e_t] * _TM + rank_t
    perm = jnp.zeros(_GROWS, jnp.int32).at[dest].set(
        jnp.arange(S, dtype=jnp.int32))
    tile_expert = jnp.minimum(
        jnp.searchsorted(tile_end, jnp.arange(P), side="right"),
        _E - 1).astype(jnp.int32)

    hk, recept, shared = pl.pallas_call(
        _pre_body,
        grid=(M, K),
        in_specs=[
            pl.BlockSpec((_TM, H), lambda m, k: (m, 0)),
            pl.BlockSpec((_TM, H), lambda m, k: (m, 0)),
            pl.BlockSpec((1, H), lambda m, k: (0, 0)),
            pl.BlockSpec((1, H), lambda m, k: (0, 0)),
            pl.BlockSpec((H, H), lambda m, k: (0, 0)),
            pl.BlockSpec((_IB, H),
                         lambda m, k: (jnp.where(m % 2 == 0, k, K - 1 - k), 0)),
            pl.BlockSpec((H, _IB),
                         lambda m, k: (0, jnp.where(m % 2 == 0, k, K - 1 - k))),
        ],
        out_specs=[
            pl.BlockSpec((_TM, H), lambda m, k: (m, 0)),
            pl.BlockSpec((_TM, H), lambda m, k: (m, 0)),
            pl.BlockSpec((_TM, H), lambda m, k: (m, 0)),
        ],
        out_shape=[jax.ShapeDtypeStruct((S, H), jnp.float32)] * 3,
    )(x, shifted, tmk, tmr, W_r, Wk_shared, Wv_shared)

    x_sorted = _sc_gather_rows(hk, perm, _GROWS, H)

    def _x_map(p, k, te):
        return (p, 0)

    def _wk_map(p, k, te):
        kk = jnp.where(p % 2 == 0, k, K - 1 - k)
        return (te[p], kk, 0)

    def _wv_map(p, k, te):
        kk = jnp.where(p % 2 == 0, k, K - 1 - k)
        return (te[p], 0, kk)

    out_sorted = pl.pallas_call(
        _grouped_body,
        grid_spec=pltpu.PrefetchScalarGridSpec(
            num_scalar_prefetch=1,
            grid=(P, K),
            in_specs=[
                pl.BlockSpec((_TM, H), _x_map),
                pl.BlockSpec((1, _IB, H), _wk_map),
                pl.BlockSpec((1, H, _IB), _wv_map),
            ],
            out_specs=pl.BlockSpec((_TM, H), _x_map),
        ),
        out_shape=jax.ShapeDtypeStruct((P * _TM, H), jnp.float32),
    )(tile_expert, x_sorted, Wk_experts, Wv_experts)

    expert_out = _sc_gather_rows(out_sorted, dest, S, H)

    out = pl.pallas_call(
        _combine_body,
        grid=(M,),
        in_specs=[pl.BlockSpec((_TM, H), lambda m: (m, 0))] * 3,
        out_specs=pl.BlockSpec((_TM, H), lambda m: (m, 0)),
        out_shape=jax.ShapeDtypeStruct((S, H), jnp.float32),
    )(recept, shared, expert_out)

    return out.reshape(B, S, H)


# speedup vs baseline: 1.3827x; 1.3827x over previous
import functools

import jax
import jax.numpy as jnp
from jax import lax
from jax.experimental import pallas as pl
from jax.experimental.pallas import tpu as pltpu
from jax.experimental.pallas import tpu_sc as plsc

_PRIME = 5099
_E = 8
_TM = 256
_MAX_TILES = 15
_GROWS = 4096
_IB = 896
_NW = 32


def _nt(a, b):
    return lax.dot_general(a.astype(jnp.bfloat16), b.astype(jnp.bfloat16),
                           (((1,), (1,)), ((), ())),
                           preferred_element_type=jnp.float32)


def _pre_body(hid_ref, sh_ref, tmk_ref, tmr_ref, wr_ref, wks_ref, wvs_ref,
              hk_ref, r_ref, s_ref):
    k = pl.program_id(1)
    hid = hid_ref[...]
    delta = sh_ref[...] - hid
    hk = hid + delta * tmk_ref[...]

    @pl.when(k == 0)
    def _():
        hk_ref[...] = hk
        rin = hid + delta * tmr_ref[...]
        r_ref[...] = jax.nn.sigmoid(_nt(rin, wr_ref[...]))

    h = jnp.maximum(_nt(hk, wks_ref[...]), 0.0)
    h = h * h
    contrib = _nt(h, wvs_ref[...])

    @pl.when(k == 0)
    def _():
        s_ref[...] = contrib

    @pl.when(k > 0)
    def _():
        s_ref[...] += contrib


def _grouped_body(te_ref, x_ref, wk_ref, wv_ref, o_ref):
    k = pl.program_id(1)
    h = jnp.maximum(_nt(x_ref[...], wk_ref[0]), 0.0)
    h = h * h
    contrib = _nt(h, wv_ref[0])

    @pl.when(k == 0)
    def _():
        o_ref[...] = contrib

    @pl.when(k > 0)
    def _():
        o_ref[...] += contrib


def _combine_body(r_ref, s_ref, eo_ref, o_ref):
    o_ref[...] = r_ref[...] * (s_ref[...] + eo_ref[...])


def _sc_gather_rows(table, idx, n_rows, d):
    bpw = n_rows // _NW
    mesh = plsc.VectorSubcoreMesh(core_axis_name="c", subcore_axis_name="s")

    @functools.partial(
        pl.kernel, mesh=mesh,
        out_type=jax.ShapeDtypeStruct((n_rows, d), jnp.float32),
        scratch_types=[
            pltpu.VMEM((bpw,), jnp.int32),
            pltpu.VMEM((bpw, d), jnp.float32),
            pltpu.SemaphoreType.DMA,
        ],
    )
    def k(table_hbm, idx_hbm, out_hbm, idx_v, rows_v, sem):
        wid = lax.axis_index("s") * 2 + lax.axis_index("c")
        base = wid * bpw
        pltpu.sync_copy(idx_hbm.at[pl.ds(base, bpw)], idx_v)
        pltpu.async_copy(table_hbm.at[idx_v], rows_v, sem).wait()
        pltpu.sync_copy(rows_v, out_hbm.at[pl.ds(base, bpw)])

    return k(table, idx)


def kernel(hidden, input_ids, time_maa_k, time_maa_r, W_r,
           Wk_shared, Wv_shared, Wk_experts, Wv_experts):
    B, S, H = hidden.shape
    I = Wk_shared.shape[0]
    K = I // _IB
    M = S // _TM
    P = _MAX_TILES

    x = hidden.reshape(S, H)
    shifted = jnp.pad(x, ((1, 0), (0, 0)))[:-1, :]
    tmk = time_maa_k.reshape(1, H)
    tmr = time_maa_r.reshape(1, H)

    ids = input_ids.reshape(-1).astype(jnp.int32)
    e_t = (ids * _PRIME) % _E
    onehot = (e_t[:, None] == jnp.arange(_E, dtype=jnp.int32)[None, :])
    ranks = jnp.cumsum(onehot.astype(jnp.int32), axis=0)
    counts = ranks[-1]
    rank_t = jnp.take_along_axis(ranks, e_t[:, None], axis=1)[:, 0] - 1
    tiles_per_e = (counts + _TM - 1) // _TM
    tile_end = jnp.cumsum(tiles_per_e)
    tile_start = tile_end - tiles_per_e
    dest = tile_start[e_t] * _TM + rank_t
    perm = (jnp.arange(_GROWS, dtype=jnp.int32) % S).at[dest].set(
        jnp.arange(S, dtype=jnp.int32))
    tile_expert = jnp.minimum(
        jnp.searchsorted(tile_end, jnp.arange(P), side="right"),
        _E - 1).astype(jnp.int32)

    hk, recept, shared = pl.pallas_call(
        _pre_body,
        grid=(M, K),
        in_specs=[
            pl.BlockSpec((_TM, H), lambda m, k: (m, 0)),
            pl.BlockSpec((_TM, H), lambda m, k: (m, 0)),
            pl.BlockSpec((1, H), lambda m, k: (0, 0)),
            pl.BlockSpec((1, H), lambda m, k: (0, 0)),
            pl.BlockSpec((H, H), lambda m, k: (0, 0)),
            pl.BlockSpec((_IB, H),
                         lambda m, k: (jnp.where(m % 2 == 0, k, K - 1 - k), 0)),
            pl.BlockSpec((H, _IB),
                         lambda m, k: (0, jnp.where(m % 2 == 0, k, K - 1 - k))),
        ],
        out_specs=[
            pl.BlockSpec((_TM, H), lambda m, k: (m, 0)),
            pl.BlockSpec((_TM, H), lambda m, k: (m, 0)),
            pl.BlockSpec((_TM, H), lambda m, k: (m, 0)),
        ],
        out_shape=[jax.ShapeDtypeStruct((S, H), jnp.float32)] * 3,
    )(x, shifted, tmk, tmr, W_r, Wk_shared, Wv_shared)

    x_sorted = _sc_gather_rows(hk, perm, _GROWS, H)

    def _x_map(p, k, te):
        return (p, 0)

    def _wk_map(p, k, te):
        kk = jnp.where(p % 2 == 0, k, K - 1 - k)
        return (te[p], kk, 0)

    def _wv_map(p, k, te):
        kk = jnp.where(p % 2 == 0, k, K - 1 - k)
        return (te[p], 0, kk)

    out_sorted = pl.pallas_call(
        _grouped_body,
        grid_spec=pltpu.PrefetchScalarGridSpec(
            num_scalar_prefetch=1,
            grid=(P, K),
            in_specs=[
                pl.BlockSpec((_TM, H), _x_map),
                pl.BlockSpec((1, _IB, H), _wk_map),
                pl.BlockSpec((1, H, _IB), _wv_map),
            ],
            out_specs=pl.BlockSpec((_TM, H), _x_map),
        ),
        out_shape=jax.ShapeDtypeStruct((P * _TM, H), jnp.float32),
    )(tile_expert, x_sorted, Wk_experts, Wv_experts)

    expert_out = _sc_gather_rows(out_sorted, dest, S, H)

    out = pl.pallas_call(
        _combine_body,
        grid=(M,),
        in_specs=[pl.BlockSpec((_TM, H), lambda m: (m, 0))] * 3,
        out_specs=pl.BlockSpec((_TM, H), lambda m: (m, 0)),
        out_shape=jax.ShapeDtypeStruct((S, H), jnp.float32),
    )(recept, shared, expert_out)

    return out.reshape(B, S, H)
